# Initial kernel scaffold; baseline (speedup 1.0000x reference)
import jax
import jax.numpy as jnp
from jax.experimental import pallas as pl


def kernel(pos, params, z, edge_index):
    raise NotImplementedError("write your pallas kernel here")


# speedup vs baseline: 2.8884x; 2.8884x over previous
import functools

import jax
import jax.numpy as jnp
from jax import lax
from jax.experimental import pallas as pl
from jax.experimental.pallas import tpu as pltpu
from jax.experimental.pallas import tpu_sc as plsc

N = 10000
NN = 10240
HID = 128
NRBF = 50
NI = 6
CUT_HI = 0.1
ALPHA = 5.0 / CUT_HI
PI = 3.14159265358979323846

NC = 2
NS = 16
NW = NC * NS
EC = 128
RPT = NN // NS

_MESH = dict(core_axis_name="c", subcore_axis_name="s")


def _silu(x):
    return x * jax.nn.sigmoid(x)


NPAD = 10240


def _sc_d2(posx, posy, posz, src, dst):
    e_pad = src.shape[0]
    epw = e_pad // NW
    nchunks = epw // EC

    @functools.partial(
        pl.kernel,
        out_type=jax.ShapeDtypeStruct((e_pad, 16), jnp.float32),
        mesh=plsc.VectorSubcoreMesh(**_MESH),
        scratch_types=[
            pltpu.VMEM((NPAD,), jnp.float32),
            pltpu.VMEM((NPAD,), jnp.float32),
            pltpu.VMEM((NPAD,), jnp.float32),
            pltpu.VMEM((EC,), jnp.int32),
            pltpu.VMEM((EC,), jnp.int32),
            pltpu.VMEM((EC, 16), jnp.float32),
        ],
        compiler_params=pltpu.CompilerParams(needs_layout_passes=False),
    )
    def k(px_hbm, py_hbm, pz_hbm, src_hbm, dst_hbm, out_hbm,
          pxv, pyv, pzv, sv, dv, pvb):
        cid = lax.axis_index("c")
        sid = lax.axis_index("s")
        base = (sid * NC + cid) * epw
        pltpu.sync_copy(px_hbm, pxv)
        pltpu.sync_copy(py_hbm, pyv)
        pltpu.sync_copy(pz_hbm, pzv)
        lanes = lax.iota(jnp.int32, 16)
        zeros16 = jnp.zeros((16,), jnp.float32)

        def zbody(e, _):
            pvb[e, :] = zeros16
            return 0

        lax.fori_loop(0, EC, zbody, 0)

        def body(i, _):
            off = base + i * EC
            pltpu.sync_copy(src_hbm.at[pl.ds(off, EC)], sv)
            pltpu.sync_copy(dst_hbm.at[pl.ds(off, EC)], dv)
            for g in range(EC // 16):
                s16 = sv[pl.ds(g * 16, 16)]
                d16 = dv[pl.ds(g * 16, 16)]
                dx = plsc.load_gather(pxv, [s16]) - plsc.load_gather(pxv, [d16])
                dy = plsc.load_gather(pyv, [s16]) - plsc.load_gather(pyv, [d16])
                dz = plsc.load_gather(pzv, [s16]) - plsc.load_gather(pzv, [d16])
                d2 = dx * dx + dy * dy + dz * dz
                rows = jnp.full((16,), g * 16, jnp.int32) + lanes
                plsc.store_scatter(pvb, [rows, jnp.zeros((16,), jnp.int32)], d2)
            pltpu.sync_copy(pvb, out_hbm.at[pl.ds(off, EC)])
            return 0

        lax.fori_loop(0, nchunks, body, 0)

    return k(posx, posy, posz, src, dst)


def _sc_gms(xtab, w, src, dst, zeros):
    e_pad = src.shape[0]
    epw = e_pad // NW
    nchunks = epw // EC

    @functools.partial(
        pl.kernel,
        out_type=jax.ShapeDtypeStruct((NC, NN, HID), jnp.float32),
        mesh=plsc.VectorSubcoreMesh(**_MESH),
        scratch_types=[
            pltpu.VMEM((EC,), jnp.int32),
            pltpu.VMEM((EC,), jnp.int32),
            pltpu.VMEM((EC, HID), jnp.float32),
            pltpu.VMEM((EC, HID), jnp.float32),
            pltpu.VMEM_SHARED((NN, HID), jnp.float32),
            pltpu.SemaphoreType.DMA,
        ],
    )
    def k(xtab_hbm, w_hbm, src_hbm, dst_hbm, z_hbm, out_hbm,
          sv, dv, xb, wb, acc, sem):
        cid = lax.axis_index("c")
        sid = lax.axis_index("s")
        base = (sid * NC + cid) * epw
        r0 = sid * RPT
        pltpu.sync_copy(z_hbm.at[pl.ds(r0, RPT)], acc.at[pl.ds(r0, RPT)])
        plsc.subcore_barrier()

        def body(i, _):
            off = base + i * EC
            pltpu.sync_copy(src_hbm.at[pl.ds(off, EC)], sv)
            pltpu.sync_copy(dst_hbm.at[pl.ds(off, EC)], dv)
            pltpu.async_copy(xtab_hbm.at[sv], xb, sem).wait()
            pltpu.sync_copy(w_hbm.at[pl.ds(off, EC)], wb)

            def ebody(e, _):
                for t in range(HID // 16):
                    sl = pl.ds(16 * t, 16)
                    xb[e, sl] = xb[e, sl] * wb[e, sl]
                return 0

            lax.fori_loop(0, EC, ebody, 0)
            pltpu.sync_copy(xb, acc.at[dv], add=True)
            return 0

        lax.fori_loop(0, nchunks, body, 0)
        plsc.subcore_barrier()
        pltpu.sync_copy(acc.at[pl.ds(r0, RPT)], out_hbm.at[cid, pl.ds(r0, RPT)])

    return k(xtab, w, src, dst, zeros)


_NB = 1024


def _tc_embed(z2, emb, ne_emb):
    def body(z_ref, emb_ref, ne_ref, h0_ref, xn_ref):
        oh = (z_ref[...] == lax.broadcasted_iota(jnp.int32, (1, 100), 1))
        oh = oh.astype(jnp.float32)
        h0_ref[...] = jnp.dot(oh, emb_ref[...], preferred_element_type=jnp.float32)
        xn_ref[...] = jnp.dot(oh, ne_ref[...], preferred_element_type=jnp.float32)

    return pl.pallas_call(
        body,
        grid=(NN // _NB,),
        in_specs=[
            pl.BlockSpec((_NB, 1), lambda i: (i, 0)),
            pl.BlockSpec((100, HID), lambda i: (0, 0)),
            pl.BlockSpec((100, HID), lambda i: (0, 0)),
        ],
        out_specs=[
            pl.BlockSpec((_NB, HID), lambda i: (i, 0)),
            pl.BlockSpec((_NB, HID), lambda i: (i, 0)),
        ],
        out_shape=[
            jax.ShapeDtypeStruct((NN, HID), jnp.float32),
            jax.ShapeDtypeStruct((NN, HID), jnp.float32),
        ],
    )(z2, emb, ne_emb)


def _tc_edgefilter(d2e, n_edges, means, betas, neT, neb, w0T, b0, m2T, b2):
    e_pad = d2e.shape[0]
    BE = 512

    def body(pv_ref, means_ref, betas_ref, neT_ref, neb_ref, w0T_ref, b0_ref,
             m2T_ref, b2_ref, wne_ref, w6_ref):
        i = pl.program_id(0)
        d2 = pv_ref[...][:, 0:1]
        ew = jnp.sqrt(d2)
        cut = 0.5 * (jnp.cos(ew * (PI / CUT_HI)) + 1.0)
        cut = jnp.where(ew < CUT_HI, cut, 0.0)
        eid = i * BE + lax.broadcasted_iota(jnp.int32, (BE, 1), 0)
        cut = jnp.where(eid < n_edges, cut, 0.0)
        expd = jnp.exp(-ALPHA * ew)
        ea = cut * jnp.exp(-betas_ref[...] * (expd - means_ref[...]) ** 2)
        wne_ref[...] = (jnp.dot(ea, neT_ref[...],
                                preferred_element_type=jnp.float32)
                        + neb_ref[...]) * cut
        h1 = _silu(jnp.dot(ea, w0T_ref[...],
                           preferred_element_type=jnp.float32) + b0_ref[...])
        for j in range(NI):
            w6_ref[j] = (jnp.dot(h1[:, 128 * j:128 * (j + 1)], m2T_ref[j],
                                 preferred_element_type=jnp.float32)
                         + b2_ref[j]) * cut

    return pl.pallas_call(
        body,
        grid=(e_pad // BE,),
        in_specs=[
            pl.BlockSpec((BE, 16), lambda i: (i, 0)),
            pl.BlockSpec((1, NRBF), lambda i: (0, 0)),
            pl.BlockSpec((1, NRBF), lambda i: (0, 0)),
            pl.BlockSpec((NRBF, HID), lambda i: (0, 0)),
            pl.BlockSpec((1, HID), lambda i: (0, 0)),
            pl.BlockSpec((NRBF, NI * HID), lambda i: (0, 0)),
            pl.BlockSpec((1, NI * HID), lambda i: (0, 0)),
            pl.BlockSpec((NI, HID, HID), lambda i: (0, 0, 0)),
            pl.BlockSpec((NI, 1, HID), lambda i: (0, 0, 0)),
        ],
        out_specs=[
            pl.BlockSpec((BE, HID), lambda i: (i, 0)),
            pl.BlockSpec((NI, BE, HID), lambda i: (0, i, 0)),
        ],
        out_shape=[
            jax.ShapeDtypeStruct((e_pad, HID), jnp.float32),
            jax.ShapeDtypeStruct((NI, e_pad, HID), jnp.float32),
        ],
    )(d2e, means, betas, neT, neb, w0T, b0, m2T, b2)


def _tc_ne_combine(h0, a0, a1, wc1T, wc2T, cb, c1T):
    def body(h0_ref, a0_ref, a1_ref, wc1_ref, wc2_ref, cb_ref, c1_ref,
             h_ref, x1_ref):
        agg = a0_ref[...] + a1_ref[...]
        h = (jnp.dot(h0_ref[...], wc1_ref[...], preferred_element_type=jnp.float32)
             + jnp.dot(agg, wc2_ref[...], preferred_element_type=jnp.float32)
             + cb_ref[...])
        h_ref[...] = h
        x1_ref[...] = jnp.dot(h, c1_ref[...], preferred_element_type=jnp.float32)

    nmat = pl.BlockSpec((_NB, HID), lambda i: (i, 0))
    wmat = pl.BlockSpec((HID, HID), lambda i: (0, 0))
    return pl.pallas_call(
        body,
        grid=(NN // _NB,),
        in_specs=[nmat, nmat, nmat, wmat, wmat,
                  pl.BlockSpec((1, HID), lambda i: (0, 0)), wmat],
        out_specs=[nmat, nmat],
        out_shape=[jax.ShapeDtypeStruct((NN, HID), jnp.float32),
                   jax.ShapeDtypeStruct((NN, HID), jnp.float32)],
    )(h0, a0, a1, wc1T, wc2T, cb, c1T)


def _tc_interact(h, a0, a1, c2T, c2b, linT, linb, c1nT):
    def body(h_ref, a0_ref, a1_ref, c2T_ref, c2b_ref, linT_ref, linb_ref,
             c1n_ref, h_ref_o, x1_ref):
        agg = a0_ref[...] + a1_ref[...]
        t = _silu(jnp.dot(agg, c2T_ref[...], preferred_element_type=jnp.float32)
                  + c2b_ref[...])
        t = jnp.dot(t, linT_ref[...], preferred_element_type=jnp.float32) + linb_ref[...]
        hn = h_ref[...] + t
        h_ref_o[...] = hn
        x1_ref[...] = jnp.dot(hn, c1n_ref[...], preferred_element_type=jnp.float32)

    nmat = pl.BlockSpec((_NB, HID), lambda i: (i, 0))
    wmat = pl.BlockSpec((HID, HID), lambda i: (0, 0))
    bvec = pl.BlockSpec((1, HID), lambda i: (0, 0))
    return pl.pallas_call(
        body,
        grid=(NN // _NB,),
        in_specs=[nmat, nmat, nmat, wmat, bvec, wmat, bvec, wmat],
        out_specs=[nmat, nmat],
        out_shape=[jax.ShapeDtypeStruct((NN, HID), jnp.float32),
                   jax.ShapeDtypeStruct((NN, HID), jnp.float32)],
    )(h, a0, a1, c2T, c2b, linT, linb, c1nT)


def _tc_readout(h, z2, o1T, o1b, o2T, o2b):
    def body(h_ref, z_ref, o1T_ref, o1b_ref, o2T_ref, o2b_ref, out_ref):
        o = _silu(jnp.dot(h_ref[...], o1T_ref[...],
                          preferred_element_type=jnp.float32) + o1b_ref[...])
        o = jnp.dot(o, o2T_ref[...], preferred_element_type=jnp.float32) + o2b_ref[...]
        o = jnp.where(z_ref[...] > 0, o, 0.0)

        @pl.when(pl.program_id(0) == 0)
        def _():
            out_ref[...] = jnp.zeros_like(out_ref)

        out_ref[...] += jnp.sum(o)

    return pl.pallas_call(
        body,
        grid=(NN // _NB,),
        in_specs=[
            pl.BlockSpec((_NB, HID), lambda i: (i, 0)),
            pl.BlockSpec((_NB, 1), lambda i: (i, 0)),
            pl.BlockSpec((HID, HID // 2), lambda i: (0, 0)),
            pl.BlockSpec((1, HID // 2), lambda i: (0, 0)),
            pl.BlockSpec((HID // 2, 1), lambda i: (0, 0)),
            pl.BlockSpec((1, 1), lambda i: (0, 0)),
        ],
        out_specs=pl.BlockSpec((1, 1), lambda i: (0, 0)),
        out_shape=jax.ShapeDtypeStruct((1, 1), jnp.float32),
    )(h, z2, o1T, o1b, o2T, o2b)


def kernel(pos, params, z, edge_index):
    p = params
    src = edge_index[0].astype(jnp.int32)
    dst = edge_index[1].astype(jnp.int32)
    n_edges = src.shape[0]
    chunk = NW * EC
    e_pad = ((n_edges + chunk - 1) // chunk) * chunk
    padn = e_pad - n_edges
    src_p = jnp.concatenate([src, jnp.zeros((padn,), jnp.int32)])
    dst_p = jnp.concatenate([dst, jnp.full((padn,), N - 1, jnp.int32)])
    posx = jnp.pad(pos[:, 0], (0, NPAD - N))
    posy = jnp.pad(pos[:, 1], (0, NPAD - N))
    posz = jnp.pad(pos[:, 2], (0, NPAD - N))
    zeros = jnp.zeros((NN, HID), jnp.float32)
    z2 = jnp.pad(z.astype(jnp.int32), (0, NN - N)).reshape(NN, 1)

    means = p['rbf_means'].reshape(1, NRBF)
    betas = p['rbf_betas'].reshape(1, NRBF)
    neT = p['ne_proj_W'].T
    neb = p['ne_proj_b'].reshape(1, HID)
    w0T = jnp.transpose(p['mlp0_W'], (2, 0, 1)).reshape(NRBF, NI * HID)
    b0 = p['mlp0_b'].reshape(1, NI * HID)
    m2T = jnp.transpose(p['mlp2_W'], (0, 2, 1))
    b2 = p['mlp2_b'].reshape(NI, 1, HID)
    wc1T = p['ne_comb_W'][:, :HID].T
    wc2T = p['ne_comb_W'][:, HID:].T
    cb = p['ne_comb_b'].reshape(1, HID)

    d2e = _sc_d2(posx, posy, posz, src_p, dst_p)
    h0, xn = _tc_embed(z2, p['emb'], p['ne_emb'])
    wne, w6 = _tc_edgefilter(d2e, n_edges, means, betas, neT, neb, w0T, b0,
                             m2T, b2)

    agg2 = _sc_gms(xn, wne, src_p, dst_p, zeros)
    h, x1 = _tc_ne_combine(h0, agg2[0], agg2[1], wc1T, wc2T, cb,
                           p['conv1_W'][0].T)

    for i in range(NI):
        agg2 = _sc_gms(x1, w6[i], src_p, dst_p, zeros)
        c1nT = p['conv1_W'][(i + 1) % NI].T
        h, x1 = _tc_interact(h, agg2[0], agg2[1],
                             p['conv2_W'][i].T, p['conv2_b'][i].reshape(1, HID),
                             p['lin_W'][i].T, p['lin_b'][i].reshape(1, HID),
                             c1nT)

    return _tc_readout(h, z2, p['out1_W'].T, p['out1_b'].reshape(1, HID // 2),
                       p['out2_W'].T, p['out2_b'].reshape(1, 1))
